# Initial kernel scaffold; baseline (speedup 1.0000x reference)
import jax
import jax.numpy as jnp
from jax.experimental import pallas as pl


def kernel(src_tokens, embed_table, condition_bias):
    raise NotImplementedError("write your pallas kernel here")


# speedup vs baseline: 1.0573x; 1.0573x over previous
import functools

import jax
import jax.numpy as jnp
from jax import lax
from jax.experimental import pallas as pl
from jax.experimental.pallas import tpu as pltpu
from jax.experimental.pallas import tpu_sc as plsc


def _build_sc_kernel(N, V, D, B, num_cores, num_subcores):
    NW = num_cores * num_subcores
    n_per_w = N // NW
    CH = 32
    NCH = n_per_w // CH
    LG = D // 16

    mesh = plsc.VectorSubcoreMesh(core_axis_name="c", subcore_axis_name="s")

    @functools.partial(
        pl.kernel,
        mesh=mesh,
        out_type=jax.ShapeDtypeStruct((N, D), jnp.float32),
        scratch_types=[
            pltpu.VMEM((n_per_w,), jnp.int32),
            pltpu.VMEM((B, D), jnp.float32),
            pltpu.VMEM((CH, D), jnp.float32),
            pltpu.SemaphoreType.DMA,
        ],
    )
    def k(tok_hbm, table_hbm, bias_hbm, out_hbm, idx_v, bias_v, rows_v, sem):
        wid = lax.axis_index("s") * num_cores + lax.axis_index("c")
        base = wid * n_per_w
        pltpu.sync_copy(tok_hbm.at[pl.ds(base, n_per_w)], idx_v)
        pltpu.sync_copy(bias_hbm, bias_v)

        def chunk_body(c, carry):
            row0 = base + c * CH
            pltpu.async_copy(
                table_hbm.at[idx_v.at[pl.ds(c * CH, CH)]], rows_v, sem
            ).wait()

            def d_body(d, dcarry):
                off = pl.multiple_of(d * 16, 16)
                bvec = [bias_v[jj, pl.ds(off, 16)] for jj in range(B)]
                for j in range(CH):
                    rows_v[j, pl.ds(off, 16)] = (
                        rows_v[j, pl.ds(off, 16)] + bvec[j % B]
                    )
                return dcarry

            lax.fori_loop(0, LG, d_body, 0)
            pltpu.sync_copy(rows_v, out_hbm.at[pl.ds(row0, CH)])
            return carry

        lax.fori_loop(0, NCH, chunk_body, 0)

    return k


def kernel(src_tokens, embed_table, condition_bias):
    S, B = src_tokens.shape
    V, D = embed_table.shape
    N = S * B
    tok = src_tokens.reshape(N).astype(jnp.int32)
    info = plsc.get_sparse_core_info()
    k = _build_sc_kernel(N, V, D, B, info.num_cores, info.num_subcores)
    out = k(tok, embed_table, condition_bias)
    return out.reshape(S, B, D)


# speedup vs baseline: 1.1446x; 1.0826x over previous
import functools

import jax
import jax.numpy as jnp
from jax import lax
from jax.experimental import pallas as pl
from jax.experimental.pallas import tpu as pltpu
from jax.experimental.pallas import tpu_sc as plsc


def _build_sc_kernel(N, V, D, B, num_cores, num_subcores):
    NW = num_cores * num_subcores
    n_per_w = N // NW
    CH = 32
    NCH = n_per_w // CH
    LG = D // 16

    mesh = plsc.VectorSubcoreMesh(core_axis_name="c", subcore_axis_name="s")

    @functools.partial(
        pl.kernel,
        mesh=mesh,
        out_type=jax.ShapeDtypeStruct((N, D), jnp.float32),
        scratch_types=[
            pltpu.VMEM((n_per_w,), jnp.int32),
            pltpu.VMEM((B, D), jnp.float32),
            pltpu.VMEM((CH, D), jnp.float32),
            pltpu.VMEM((CH, D), jnp.float32),
            pltpu.SemaphoreType.DMA,
            pltpu.SemaphoreType.DMA,
            pltpu.SemaphoreType.DMA,
            pltpu.SemaphoreType.DMA,
        ],
    )
    def k(tok_hbm, table_hbm, bias_hbm, out_hbm,
          idx_v, bias_v, rows0, rows1, g0, g1, s0, s1):
        wid = lax.axis_index("s") * num_cores + lax.axis_index("c")
        base = wid * n_per_w
        pltpu.sync_copy(tok_hbm.at[pl.ds(base, n_per_w)], idx_v)
        pltpu.sync_copy(bias_hbm, bias_v)

        rows = (rows0, rows1)
        gsem = (g0, g1)
        ssem = (s0, s1)

        def gather(c, b):
            return pltpu.make_async_copy(
                table_hbm.at[idx_v.at[pl.ds(c * CH, CH)]], rows[b], gsem[b]
            )

        def scatter(c, b):
            return pltpu.make_async_copy(
                rows[b], out_hbm.at[pl.ds(base + c * CH, CH)], ssem[b]
            )

        def add_bias(b):
            rv = rows[b]

            def d_body(d, dcarry):
                off = pl.multiple_of(d * 16, 16)
                bvec = [bias_v[jj, pl.ds(off, 16)] for jj in range(B)]
                for j in range(CH):
                    rv[j, pl.ds(off, 16)] = rv[j, pl.ds(off, 16)] + bvec[j % B]
                return dcarry

            lax.fori_loop(0, LG, d_body, 0)

        gather(0, 0).start()

        def pair_body(i, carry):
            cc0 = 2 * i
            gather(cc0, 0).wait()
            add_bias(0)
            scatter(cc0, 0).start()

            @pl.when(i >= 1)
            def _():
                scatter(cc0 - 1, 1).wait()

            gather(cc0 + 1, 1).start()

            gather(cc0 + 1, 1).wait()
            add_bias(1)
            scatter(cc0 + 1, 1).start()
            scatter(cc0, 0).wait()

            @pl.when(i < NCH // 2 - 1)
            def _():
                gather(cc0 + 2, 0).start()

            return carry

        lax.fori_loop(0, NCH // 2, pair_body, 0)
        scatter(NCH - 1, 1).wait()

    return k


def kernel(src_tokens, embed_table, condition_bias):
    S, B = src_tokens.shape
    V, D = embed_table.shape
    N = S * B
    tok = src_tokens.reshape(N).astype(jnp.int32)
    info = plsc.get_sparse_core_info()
    k = _build_sc_kernel(N, V, D, B, info.num_cores, info.num_subcores)
    out = k(tok, embed_table, condition_bias)
    return out.reshape(S, B, D)


# speedup vs baseline: 1.3267x; 1.1591x over previous
import functools

import jax
import jax.numpy as jnp
from jax import lax
from jax.experimental import pallas as pl
from jax.experimental.pallas import tpu as pltpu
from jax.experimental.pallas import tpu_sc as plsc


def _build_sc_kernel(N, V, D, B, num_cores, num_subcores):
    NW = num_cores * num_subcores
    n_per_w = N // NW
    CH = 16
    NB = 4
    NCH = n_per_w // CH
    LG = D // 16

    mesh = plsc.VectorSubcoreMesh(core_axis_name="c", subcore_axis_name="s")

    @functools.partial(
        pl.kernel,
        mesh=mesh,
        out_type=jax.ShapeDtypeStruct((N, D), jnp.float32),
        scratch_types=[
            pltpu.VMEM((n_per_w,), jnp.int32),
            pltpu.VMEM((B, D), jnp.float32),
        ]
        + [pltpu.VMEM((CH, D), jnp.float32)] * NB
        + [pltpu.SemaphoreType.DMA] * (2 * NB),
    )
    def k(tok_hbm, table_hbm, bias_hbm, out_hbm, idx_v, bias_v, *bufs):
        rows = bufs[:NB]
        gsem = bufs[NB:2 * NB]
        ssem = bufs[2 * NB:]
        wid = lax.axis_index("s") * num_cores + lax.axis_index("c")
        base = wid * n_per_w
        pltpu.sync_copy(tok_hbm.at[pl.ds(base, n_per_w)], idx_v)
        pltpu.sync_copy(bias_hbm, bias_v)

        def gather(c, b):
            return pltpu.make_async_copy(
                table_hbm.at[idx_v.at[pl.ds(c * CH, CH)]], rows[b], gsem[b]
            )

        def scatter(c, b):
            return pltpu.make_async_copy(
                rows[b], out_hbm.at[pl.ds(base + c * CH, CH)], ssem[b]
            )

        def add_bias(b):
            rv = rows[b]

            def d_body(d, dcarry):
                off = pl.multiple_of(d * 16, 16)
                bvec = [bias_v[jj, pl.ds(off, 16)] for jj in range(B)]
                for j in range(CH):
                    rv[j, pl.ds(off, 16)] = rv[j, pl.ds(off, 16)] + bvec[j % B]
                return dcarry

            lax.fori_loop(0, LG, d_body, 0)

        for c in range(NB):
            gather(c, c).start()

        def ring_body(i, carry):
            for b in range(NB):
                cc = NB * i + b
                gather(cc, b).wait()
                add_bias(b)
                scatter(cc, b).start()

                pb = (b - 1) % NB

                def refill(cc=cc, pb=pb):
                    scatter(cc - 1, pb).wait()
                    gather(cc + NB - 1, pb).start()

                if b == 0:
                    pl.when(i >= 1)(refill)
                else:
                    pl.when(i < NCH // NB - 1)(refill)
            return carry

        lax.fori_loop(0, NCH // NB, ring_body, 0)
        for c in range(NCH - NB, NCH):
            scatter(c, c % NB).wait()

    return k


def kernel(src_tokens, embed_table, condition_bias):
    S, B = src_tokens.shape
    V, D = embed_table.shape
    N = S * B
    tok = src_tokens.reshape(N).astype(jnp.int32)
    info = plsc.get_sparse_core_info()
    k = _build_sc_kernel(N, V, D, B, info.num_cores, info.num_subcores)
    out = k(tok, embed_table, condition_bias)
    return out.reshape(S, B, D)
